# baseline (speedup 1.0000x reference)
import jax
import jax.numpy as jnp
from jax.experimental import pallas as pl


def _edge_diff(new_e, old_e, N):
    pn = jnp.zeros((N, N), jnp.bool_).at[new_e[0], new_e[1]].set(True)
    po = jnp.zeros((N, N), jnp.bool_).at[old_e[0], old_e[1]].set(True)
    return (pn ^ po).astype(jnp.float32)


def _indirect_counts(Ap, An, M):
    pos = jnp.zeros((M, M), jnp.bool_).at[Ap[0], Ap[1]].set(True)
    neg = jnp.zeros((M, M), jnp.bool_).at[An[0], An[1]].set(True)
    eye = jnp.eye(M, dtype=jnp.bool_)
    A1 = jnp.where(eye, 0.0, pos.astype(jnp.float32))
    A2 = jnp.where(eye, 0.0, (neg & ~pos).astype(jnp.float32))
    n_pos = jnp.rint(A1 @ A1 + A2 @ A2)
    n_neg = jnp.rint(A1 @ A2 + A2 @ A1)
    upper = jnp.triu(jnp.ones((M, M), jnp.bool_), 1)
    n_pos = jnp.where(upper & ~pos, n_pos, 0.0)
    n_neg = jnp.where(upper & ~neg, n_neg, 0.0)
    return n_pos, n_neg


def _gcn_init(x, src, dst, W, b):
    N = x.shape[0]
    loop = jnp.arange(N, dtype=jnp.int32)
    s = jnp.concatenate([src.astype(jnp.int32), loop])
    d = jnp.concatenate([dst.astype(jnp.int32), loop])
    deg = jnp.zeros((N,), jnp.float32).at[d].add(1.0)
    dinv = jnp.where(deg > 0, deg ** -0.5, 0.0)
    norm = dinv[s] * dinv[d]
    h = x @ W
    out = jnp.zeros((N, W.shape[1]), jnp.float32).at[d].add(norm[:, None] * h[s])
    return out + b


def _mp_step_kernel(CT_ref, hk_ref, hi_ref, dinv_ref, b_ref, tf_ref, out_ref):
    k = pl.program_id(1)
    nk = pl.num_programs(1)
    part = jnp.dot(CT_ref[...], hk_ref[...], preferred_element_type=jnp.float32)

    @pl.when(k == 0)
    def _():
        out_ref[...] = part

    @pl.when(k != 0)
    def _():
        out_ref[...] = out_ref[...] + part

    @pl.when(k == nk - 1)
    def _():
        acc = out_ref[...] + hi_ref[...]
        y = dinv_ref[...] * acc + b_ref[...]
        out_ref[...] = jnp.maximum(y, 0.0) * tf_ref[...]


def _mp_step(CT_pad, h, dinv_pad, b2, tf):
    PN = CT_pad.shape[0]
    H = h.shape[1]
    BI, BK = 256, 2048
    return pl.pallas_call(
        _mp_step_kernel,
        grid=(PN // BI, PN // BK),
        in_specs=[
            pl.BlockSpec((BI, BK), lambda i, k: (i, k)),
            pl.BlockSpec((BK, H), lambda i, k: (k, 0)),
            pl.BlockSpec((BI, H), lambda i, k: (i, 0)),
            pl.BlockSpec((BI, 1), lambda i, k: (i, 0)),
            pl.BlockSpec((1, H), lambda i, k: (0, 0)),
            pl.BlockSpec((1, H), lambda i, k: (0, 0)),
        ],
        out_specs=pl.BlockSpec((BI, H), lambda i, k: (i, 0)),
        out_shape=jax.ShapeDtypeStruct((PN, H), jnp.float32),
    )(CT_pad, h, h, dinv_pad, b2, tf)


def _rk4_pallas(x0, C, Wt, W, b):
    N = C.shape[0]
    PN = ((N + 1023) // 1024) * 1024
    CT_pad = jnp.zeros((PN, PN), jnp.float32).at[:N, :N].set(C.T)
    deg = C.sum(axis=0) + 1.0
    dinv = jnp.where(deg > 0, deg ** -0.5, 0.0)
    dinv_pad = jnp.zeros((PN, 1), jnp.float32).at[:N, 0].set(dinv)
    b2 = b.reshape(1, -1)

    def f(t, x):
        tf = jax.nn.sigmoid(jnp.full((1, 1), t, jnp.float32) @ Wt.T)
        h = dinv_pad * (x @ W)
        return _mp_step(CT_pad, h, dinv_pad, b2, tf)

    x = jnp.zeros((PN, x0.shape[1]), jnp.float32).at[:N].set(x0)
    h = 0.1
    t = 0.0
    for _ in range(10):
        k1 = f(t, x)
        k2 = f(t + h / 2, x + h / 2 * k1)
        k3 = f(t + h / 2, x + h / 2 * k2)
        k4 = f(t + h, x + h * k3)
        x = x + (h / 6.0) * (k1 + 2 * k2 + 2 * k3 + k4)
        t += h
    return x[:N]


def kernel(H_t, A_pos_t, A_pos_tp1, A_neg_t, A_neg_tp1, W_init, b_init, Wt_pos, W_pos, b_pos, Wt_neg, W_neg, b_neg, W_comb, b_comb, gamma, beta):
    N = H_t.shape[0]
    M = 2048
    e0 = jnp.concatenate([A_pos_t, A_neg_t], axis=1)
    H1 = jax.nn.relu(_gcn_init(H_t, e0[0], e0[1], W_init, b_init))

    d_pos = _edge_diff(A_pos_tp1, A_pos_t, N)
    d_neg = _edge_diff(A_neg_tp1, A_neg_t, N)
    n_pos, n_neg = _indirect_counts(A_pos_tp1, A_neg_tp1, M)
    C_pos = d_pos.at[:M, :M].add(n_pos)
    C_neg = d_neg.at[:M, :M].add(n_neg)

    z_pos = jnp.where(C_pos.sum() > 0, _rk4_pallas(H1, C_pos, Wt_pos, W_pos, b_pos), jnp.zeros_like(H1))
    z_neg = jnp.where(C_neg.sum() > 0, _rk4_pallas(H1, C_neg, Wt_neg, W_neg, b_neg), jnp.zeros_like(H1))

    z = jnp.concatenate([z_pos, z_neg], axis=-1) @ W_comb + b_comb
    mu = z.mean(-1, keepdims=True)
    var = ((z - mu) ** 2).mean(-1, keepdims=True)
    return (z - mu) / jnp.sqrt(var + 1e-5) * gamma + beta


# speedup vs baseline: 1.2106x; 1.2106x over previous
import jax
import jax.numpy as jnp
from jax.experimental import pallas as pl


def _edge_diff(new_e, old_e, N):
    pn = jnp.zeros((N, N), jnp.bool_).at[new_e[0], new_e[1]].set(True)
    po = jnp.zeros((N, N), jnp.bool_).at[old_e[0], old_e[1]].set(True)
    return (pn ^ po).astype(jnp.float32)


def _indirect_counts(Ap, An, M):
    pos = jnp.zeros((M, M), jnp.bool_).at[Ap[0], Ap[1]].set(True)
    neg = jnp.zeros((M, M), jnp.bool_).at[An[0], An[1]].set(True)
    eye = jnp.eye(M, dtype=jnp.bool_)
    A1 = jnp.where(eye, 0.0, pos.astype(jnp.float32))
    A2 = jnp.where(eye, 0.0, (neg & ~pos).astype(jnp.float32))
    n_pos = jnp.rint(A1 @ A1 + A2 @ A2)
    n_neg = jnp.rint(A1 @ A2 + A2 @ A1)
    upper = jnp.triu(jnp.ones((M, M), jnp.bool_), 1)
    n_pos = jnp.where(upper & ~pos, n_pos, 0.0)
    n_neg = jnp.where(upper & ~neg, n_neg, 0.0)
    return n_pos, n_neg


def _gcn_init(x, src, dst, W, b):
    N = x.shape[0]
    loop = jnp.arange(N, dtype=jnp.int32)
    s = jnp.concatenate([src.astype(jnp.int32), loop])
    d = jnp.concatenate([dst.astype(jnp.int32), loop])
    deg = jnp.zeros((N,), jnp.float32).at[d].add(1.0)
    dinv = jnp.where(deg > 0, deg ** -0.5, 0.0)
    norm = dinv[s] * dinv[d]
    h = x @ W
    out = jnp.zeros((N, W.shape[1]), jnp.float32).at[d].add(norm[:, None] * h[s])
    return out + b


def _mp_step_kernel(CT_ref, hk_ref, hi_ref, dinv_ref, b_ref, tf_ref, out_ref):
    k = pl.program_id(1)
    nk = pl.num_programs(1)
    part = jnp.dot(CT_ref[...], hk_ref[...], preferred_element_type=jnp.float32)

    @pl.when(k == 0)
    def _():
        out_ref[...] = part

    @pl.when(k != 0)
    def _():
        out_ref[...] = out_ref[...] + part

    @pl.when(k == nk - 1)
    def _():
        acc = out_ref[...] + hi_ref[...]
        y = dinv_ref[...] * acc + b_ref[...]
        out_ref[...] = jnp.maximum(y, 0.0) * tf_ref[...]


def _mp_step(CT_pad, h, dinv_pad, b2, tf):
    PN = CT_pad.shape[0]
    H = h.shape[1]
    BI, BK = 512, 2048
    return pl.pallas_call(
        _mp_step_kernel,
        grid=(PN // BI, PN // BK),
        in_specs=[
            pl.BlockSpec((BI, BK), lambda i, k: (i, k)),
            pl.BlockSpec((BK, H), lambda i, k: (k, 0)),
            pl.BlockSpec((BI, H), lambda i, k: (i, 0)),
            pl.BlockSpec((BI, 1), lambda i, k: (i, 0)),
            pl.BlockSpec((1, H), lambda i, k: (0, 0)),
            pl.BlockSpec((1, H), lambda i, k: (0, 0)),
        ],
        out_specs=pl.BlockSpec((BI, H), lambda i, k: (i, 0)),
        out_shape=jax.ShapeDtypeStruct((PN, H), jnp.float32),
    )(CT_pad, h, h, dinv_pad, b2, tf)


def _rk4_pallas(x0, C, Wt, W, b):
    N = C.shape[0]
    PN = ((N + 1023) // 1024) * 1024
    CT_pad = jnp.zeros((PN, PN), jnp.float32).at[:N, :N].set(C.T)
    deg = C.sum(axis=0) + 1.0
    dinv = jnp.where(deg > 0, deg ** -0.5, 0.0)
    dinv_pad = jnp.zeros((PN, 1), jnp.float32).at[:N, 0].set(dinv)
    b2 = b.reshape(1, -1)

    def f(t, x):
        tf = jax.nn.sigmoid(jnp.full((1, 1), t, jnp.float32) @ Wt.T)
        h = dinv_pad * (x @ W)
        return _mp_step(CT_pad, h, dinv_pad, b2, tf)

    x = jnp.zeros((PN, x0.shape[1]), jnp.float32).at[:N].set(x0)
    h = 0.1
    t = 0.0
    for _ in range(10):
        k1 = f(t, x)
        k2 = f(t + h / 2, x + h / 2 * k1)
        k3 = f(t + h / 2, x + h / 2 * k2)
        k4 = f(t + h, x + h * k3)
        x = x + (h / 6.0) * (k1 + 2 * k2 + 2 * k3 + k4)
        t += h
    return x[:N]


def kernel(H_t, A_pos_t, A_pos_tp1, A_neg_t, A_neg_tp1, W_init, b_init, Wt_pos, W_pos, b_pos, Wt_neg, W_neg, b_neg, W_comb, b_comb, gamma, beta):
    N = H_t.shape[0]
    M = 2048
    e0 = jnp.concatenate([A_pos_t, A_neg_t], axis=1)
    H1 = jax.nn.relu(_gcn_init(H_t, e0[0], e0[1], W_init, b_init))

    d_pos = _edge_diff(A_pos_tp1, A_pos_t, N)
    d_neg = _edge_diff(A_neg_tp1, A_neg_t, N)
    n_pos, n_neg = _indirect_counts(A_pos_tp1, A_neg_tp1, M)
    C_pos = d_pos.at[:M, :M].add(n_pos)
    C_neg = d_neg.at[:M, :M].add(n_neg)

    z_pos = jnp.where(C_pos.sum() > 0, _rk4_pallas(H1, C_pos, Wt_pos, W_pos, b_pos), jnp.zeros_like(H1))
    z_neg = jnp.where(C_neg.sum() > 0, _rk4_pallas(H1, C_neg, Wt_neg, W_neg, b_neg), jnp.zeros_like(H1))

    z = jnp.concatenate([z_pos, z_neg], axis=-1) @ W_comb + b_comb
    mu = z.mean(-1, keepdims=True)
    var = ((z - mu) ** 2).mean(-1, keepdims=True)
    return (z - mu) / jnp.sqrt(var + 1e-5) * gamma + beta


# speedup vs baseline: 1.3318x; 1.1001x over previous
import jax
import jax.numpy as jnp
from jax.experimental import pallas as pl


def _edge_diff(new_e, old_e, N):
    pn = jnp.zeros((N, N), jnp.bool_).at[new_e[0], new_e[1]].set(True)
    po = jnp.zeros((N, N), jnp.bool_).at[old_e[0], old_e[1]].set(True)
    return (pn ^ po).astype(jnp.float32)


def _indirect_counts(Ap, An, M):
    pos = jnp.zeros((M, M), jnp.bool_).at[Ap[0], Ap[1]].set(True)
    neg = jnp.zeros((M, M), jnp.bool_).at[An[0], An[1]].set(True)
    eye = jnp.eye(M, dtype=jnp.bool_)
    A1 = jnp.where(eye, 0.0, pos.astype(jnp.float32))
    A2 = jnp.where(eye, 0.0, (neg & ~pos).astype(jnp.float32))
    n_pos = jnp.rint(A1 @ A1 + A2 @ A2)
    n_neg = jnp.rint(A1 @ A2 + A2 @ A1)
    upper = jnp.triu(jnp.ones((M, M), jnp.bool_), 1)
    n_pos = jnp.where(upper & ~pos, n_pos, 0.0)
    n_neg = jnp.where(upper & ~neg, n_neg, 0.0)
    return n_pos, n_neg


def _gcn_init(x, src, dst, W, b):
    N = x.shape[0]
    loop = jnp.arange(N, dtype=jnp.int32)
    s = jnp.concatenate([src.astype(jnp.int32), loop])
    d = jnp.concatenate([dst.astype(jnp.int32), loop])
    deg = jnp.zeros((N,), jnp.float32).at[d].add(1.0)
    dinv = jnp.where(deg > 0, deg ** -0.5, 0.0)
    norm = dinv[s] * dinv[d]
    h = x @ W
    out = jnp.zeros((N, W.shape[1]), jnp.float32).at[d].add(norm[:, None] * h[s])
    return out + b


def _mp_step_kernel(CT_ref, hk_ref, hi_ref, dinv_ref, b_ref, tf_ref, out_ref):
    k = pl.program_id(1)
    nk = pl.num_programs(1)
    part = jnp.dot(CT_ref[...], hk_ref[...], preferred_element_type=jnp.float32)

    @pl.when(k == 0)
    def _():
        out_ref[...] = part

    @pl.when(k != 0)
    def _():
        out_ref[...] = out_ref[...] + part

    @pl.when(k == nk - 1)
    def _():
        acc = out_ref[...] + hi_ref[...]
        y = dinv_ref[...] * acc + b_ref[...]
        out_ref[...] = jnp.maximum(y, 0.0) * tf_ref[...]


def _mp_step(CT_pad, h, dinv_pad, b2, tf):
    PN = CT_pad.shape[0]
    H = h.shape[1]
    BI, BK = 1024, 2048
    return pl.pallas_call(
        _mp_step_kernel,
        grid=(PN // BI, PN // BK),
        in_specs=[
            pl.BlockSpec((BI, BK), lambda i, k: (i, k)),
            pl.BlockSpec((BK, H), lambda i, k: (k, 0)),
            pl.BlockSpec((BI, H), lambda i, k: (i, 0)),
            pl.BlockSpec((BI, 1), lambda i, k: (i, 0)),
            pl.BlockSpec((1, H), lambda i, k: (0, 0)),
            pl.BlockSpec((1, H), lambda i, k: (0, 0)),
        ],
        out_specs=pl.BlockSpec((BI, H), lambda i, k: (i, 0)),
        out_shape=jax.ShapeDtypeStruct((PN, H), jnp.float32),
    )(CT_pad, h, h, dinv_pad, b2, tf)


def _rk4_pallas(x0, C, Wt, W, b):
    N = C.shape[0]
    PN = ((N + 1023) // 1024) * 1024
    CT_pad = jnp.zeros((PN, PN), jnp.float32).at[:N, :N].set(C.T)
    deg = C.sum(axis=0) + 1.0
    dinv = jnp.where(deg > 0, deg ** -0.5, 0.0)
    dinv_pad = jnp.zeros((PN, 1), jnp.float32).at[:N, 0].set(dinv)
    b2 = b.reshape(1, -1)

    def f(t, x):
        tf = jax.nn.sigmoid(jnp.full((1, 1), t, jnp.float32) @ Wt.T)
        h = dinv_pad * (x @ W)
        return _mp_step(CT_pad, h, dinv_pad, b2, tf)

    x = jnp.zeros((PN, x0.shape[1]), jnp.float32).at[:N].set(x0)
    h = 0.1
    t = 0.0
    for _ in range(10):
        k1 = f(t, x)
        k2 = f(t + h / 2, x + h / 2 * k1)
        k3 = f(t + h / 2, x + h / 2 * k2)
        k4 = f(t + h, x + h * k3)
        x = x + (h / 6.0) * (k1 + 2 * k2 + 2 * k3 + k4)
        t += h
    return x[:N]


def kernel(H_t, A_pos_t, A_pos_tp1, A_neg_t, A_neg_tp1, W_init, b_init, Wt_pos, W_pos, b_pos, Wt_neg, W_neg, b_neg, W_comb, b_comb, gamma, beta):
    N = H_t.shape[0]
    M = 2048
    e0 = jnp.concatenate([A_pos_t, A_neg_t], axis=1)
    H1 = jax.nn.relu(_gcn_init(H_t, e0[0], e0[1], W_init, b_init))

    d_pos = _edge_diff(A_pos_tp1, A_pos_t, N)
    d_neg = _edge_diff(A_neg_tp1, A_neg_t, N)
    n_pos, n_neg = _indirect_counts(A_pos_tp1, A_neg_tp1, M)
    C_pos = d_pos.at[:M, :M].add(n_pos)
    C_neg = d_neg.at[:M, :M].add(n_neg)

    z_pos = jnp.where(C_pos.sum() > 0, _rk4_pallas(H1, C_pos, Wt_pos, W_pos, b_pos), jnp.zeros_like(H1))
    z_neg = jnp.where(C_neg.sum() > 0, _rk4_pallas(H1, C_neg, Wt_neg, W_neg, b_neg), jnp.zeros_like(H1))

    z = jnp.concatenate([z_pos, z_neg], axis=-1) @ W_comb + b_comb
    mu = z.mean(-1, keepdims=True)
    var = ((z - mu) ** 2).mean(-1, keepdims=True)
    return (z - mu) / jnp.sqrt(var + 1e-5) * gamma + beta
